# Initial kernel scaffold; baseline (speedup 1.0000x reference)
import jax
import jax.numpy as jnp
from jax import lax
from jax.experimental import pallas as pl
from jax.experimental.pallas import tpu as pltpu
from jax.experimental.pallas import tpu_sc as plsc

H = 128
L = 16
NC = 2
NS = 16
NW = NC * NS
CHUNK = 128
BN_EPS = 1e-3


def _make_sc_segment_sum(n_pad, k_chunks, compute_deg):
    rows_per_tile = n_pad // NS
    mesh = plsc.VectorSubcoreMesh(core_axis_name="c", subcore_axis_name="s")

    out_type = [jax.ShapeDtypeStruct((NC, n_pad, H), jnp.float32)]
    scratch = [
        pltpu.VMEM((CHUNK,), jnp.int32),
        pltpu.VMEM((CHUNK,), jnp.int32),
        pltpu.VMEM((CHUNK, H), jnp.float32),
        pltpu.VMEM_SHARED((n_pad, H), jnp.float32),
        pltpu.SemaphoreType.DMA,
    ]
    if compute_deg:
        out_type.append(jax.ShapeDtypeStruct((NC, n_pad, L), jnp.float32))
        scratch += [
            pltpu.VMEM((CHUNK, L), jnp.float32),
            pltpu.VMEM_SHARED((n_pad, L), jnp.float32),
        ]

    if compute_deg:
        def body(h, src_i, dst_i, zeros_s, zeros_d, ones,
                 s_out, deg_out, isrc, idst, rows, s_sh, sem, ones_v, deg_sh):
            _sc_body(h, src_i, dst_i, zeros_s, zeros_d, ones, s_out, deg_out,
                     isrc, idst, rows, s_sh, sem, ones_v, deg_sh,
                     k_chunks, rows_per_tile, True)
    else:
        def body(h, src_i, dst_i, zeros_s,
                 s_out, isrc, idst, rows, s_sh, sem):
            _sc_body(h, src_i, dst_i, zeros_s, None, None, s_out, None,
                     isrc, idst, rows, s_sh, sem, None, None,
                     k_chunks, rows_per_tile, False)

    return pl.kernel(body, out_type=tuple(out_type), mesh=mesh,
                     scratch_types=tuple(scratch))


def _sc_body(h, src_i, dst_i, zeros_s, zeros_d, ones, s_out, deg_out,
             isrc, idst, rows, s_sh, sem, ones_v, deg_sh,
             k_chunks, rows_per_tile, compute_deg):
    c = lax.axis_index("c")
    s = lax.axis_index("s")
    wid = s * NC + c
    row0 = s * rows_per_tile

    pltpu.sync_copy(zeros_s.at[pl.ds(row0, rows_per_tile)],
                    s_sh.at[pl.ds(row0, rows_per_tile)])
    if compute_deg:
        pltpu.sync_copy(zeros_d.at[pl.ds(row0, rows_per_tile)],
                        deg_sh.at[pl.ds(row0, rows_per_tile)])
        pltpu.sync_copy(ones, ones_v)
    plsc.subcore_barrier()

    def step(j, carry):
        pltpu.sync_copy(src_i.at[wid, j], isrc)
        pltpu.sync_copy(dst_i.at[wid, j], idst)
        pltpu.async_copy(h.at[isrc], rows, sem).wait()
        pltpu.sync_copy(rows, s_sh.at[idst], add=True)
        if compute_deg:
            pltpu.sync_copy(ones_v, deg_sh.at[idst], add=True)
        return carry

    lax.fori_loop(0, k_chunks, step, 0)
    plsc.subcore_barrier()

    pltpu.sync_copy(s_sh.at[pl.ds(row0, rows_per_tile)],
                    s_out.at[c, pl.ds(row0, rows_per_tile)])
    if compute_deg:
        pltpu.sync_copy(deg_sh.at[pl.ds(row0, rows_per_tile)],
                        deg_out.at[c, pl.ds(row0, rows_per_tile)])


def _emb_body(x_ref, w_ref, b_ref, o_ref):
    o_ref[...] = jnp.dot(x_ref[...], w_ref[...],
                         preferred_element_type=jnp.float32,
                         precision=lax.Precision.HIGHEST) + b_ref[...]


def _embed(x, w, b, blk):
    n = x.shape[0]
    return pl.pallas_call(
        _emb_body,
        grid=(n // blk,),
        in_specs=[
            pl.BlockSpec((blk, H), lambda i: (i, 0)),
            pl.BlockSpec((H, H), lambda i: (0, 0)),
            pl.BlockSpec((1, H), lambda i: (0, 0)),
        ],
        out_specs=pl.BlockSpec((blk, H), lambda i: (i, 0)),
        out_shape=jax.ShapeDtypeStruct((n, H), jnp.float32),
    )(x, w, b.reshape(1, H))


def _layer_body(h_ref, s_ref, deg_ref, wt_ref, wb_ref, p_ref, o_ref):
    h = h_ref[...]
    s_sum = s_ref[0] + s_ref[1]
    deg = deg_ref[0, :, 0:1] + deg_ref[1, :, 0:1]
    b = p_ref[0:1, :]
    gamma = p_ref[1:2, :]
    beta = p_ref[2:3, :]
    mean = p_ref[3:4, :]
    var = p_ref[4:5, :]
    hw = jnp.dot(h, wb_ref[...], preferred_element_type=jnp.float32,
                 precision=lax.Precision.HIGHEST)
    agg = jnp.dot(s_sum, wt_ref[...], preferred_element_type=jnp.float32,
                  precision=lax.Precision.HIGHEST) + deg * (hw + b)
    z = jax.nn.sigmoid(agg) + jax.nn.softplus(h)
    z = (z - mean) / jnp.sqrt(var + BN_EPS) * gamma + beta
    o_ref[...] = jnp.maximum(z, 0.0)


def _layer_dense(h, s_partial, deg_partial, w, params, blk):
    n = h.shape[0]
    return pl.pallas_call(
        _layer_body,
        grid=(n // blk,),
        in_specs=[
            pl.BlockSpec((blk, H), lambda i: (i, 0)),
            pl.BlockSpec((NC, blk, H), lambda i: (0, i, 0)),
            pl.BlockSpec((NC, blk, L), lambda i: (0, i, 0)),
            pl.BlockSpec((H, H), lambda i: (0, 0)),
            pl.BlockSpec((H, H), lambda i: (0, 0)),
            pl.BlockSpec((5, H), lambda i: (0, 0)),
        ],
        out_specs=pl.BlockSpec((blk, H), lambda i: (i, 0)),
        out_shape=jax.ShapeDtypeStruct((n, H), jnp.float32),
    )(h, s_partial, deg_partial, w[:H], w[H:], params)


def kernel(node_feat, edge_index, W_emb, b_emb,
           w0, b0, gamma0, beta0, mean0, var0,
           w1, b1, gamma1, beta1, mean1, var1):
    n = node_feat.shape[0]
    e = edge_index.shape[1]
    n_pad = ((n + 1 + NS - 1) // NS) * NS
    e_pad = ((e + NW * CHUNK - 1) // (NW * CHUNK)) * (NW * CHUNK)
    k_chunks = e_pad // (NW * CHUNK)

    src = edge_index[0].astype(jnp.int32)
    dst = edge_index[1].astype(jnp.int32)
    src_i = jnp.zeros((e_pad,), jnp.int32).at[:e].set(src)
    dst_i = jnp.full((e_pad,), n, jnp.int32).at[:e].set(dst)
    src_i = src_i.reshape(NW, k_chunks, CHUNK)
    dst_i = dst_i.reshape(NW, k_chunks, CHUNK)

    zeros_s = jnp.zeros((n_pad, H), jnp.float32)
    zeros_d = jnp.zeros((n_pad, L), jnp.float32)
    ones = jnp.ones((CHUNK, L), jnp.float32)

    sc_first = _make_sc_segment_sum(n_pad, k_chunks, compute_deg=True)
    sc_next = _make_sc_segment_sum(n_pad, k_chunks, compute_deg=False)

    blk = 2000
    h = _embed(node_feat, W_emb, b_emb, blk)

    s_p, deg_p = sc_first(h, src_i, dst_i, zeros_s, zeros_d, ones)
    params0 = jnp.stack([b0, gamma0, beta0, mean0, var0])
    h = _layer_dense(h, s_p, deg_p, w0, params0, blk)

    (s_p,) = sc_next(h, src_i, dst_i, zeros_s)
    params1 = jnp.stack([b1, gamma1, beta1, mean1, var1])
    h = _layer_dense(h, s_p, deg_p, w1, params1, blk)

    return h


# speedup vs baseline: 5.2937x; 5.2937x over previous
import jax
import jax.numpy as jnp
from jax import lax
from jax.experimental import pallas as pl
from jax.experimental.pallas import tpu as pltpu
from jax.experimental.pallas import tpu_sc as plsc

H = 128
L = 16
NC = 2
NS = 16
NW = NC * NS
CHUNK = 128
BN_EPS = 1e-3


def _make_sc_segment_sum(n_pad, k_chunks, compute_deg):
    rows_per_tile = n_pad // NS
    mesh = plsc.VectorSubcoreMesh(core_axis_name="c", subcore_axis_name="s")

    out_type = [jax.ShapeDtypeStruct((NC, n_pad, H), jnp.float32)]
    scratch = [
        pltpu.VMEM((CHUNK,), jnp.int32),
        pltpu.VMEM((CHUNK,), jnp.int32),
        pltpu.VMEM((CHUNK, H), jnp.float32),
        pltpu.VMEM_SHARED((n_pad, H), jnp.float32),
        pltpu.SemaphoreType.DMA,
    ]
    if compute_deg:
        out_type.append(jax.ShapeDtypeStruct((NC, n_pad, L), jnp.float32))
        scratch += [
            pltpu.VMEM((CHUNK, L), jnp.float32),
            pltpu.VMEM_SHARED((n_pad, L), jnp.float32),
        ]

    if compute_deg:
        def body(h, src_i, dst_i, zeros_s, zeros_d, ones,
                 s_out, deg_out, isrc, idst, rows, s_sh, sem, ones_v, deg_sh):
            _sc_body(h, src_i, dst_i, zeros_s, zeros_d, ones, s_out, deg_out,
                     isrc, idst, rows, s_sh, sem, ones_v, deg_sh,
                     k_chunks, rows_per_tile, True)
    else:
        def body(h, src_i, dst_i, zeros_s,
                 s_out, isrc, idst, rows, s_sh, sem):
            _sc_body(h, src_i, dst_i, zeros_s, None, None, s_out, None,
                     isrc, idst, rows, s_sh, sem, None, None,
                     k_chunks, rows_per_tile, False)

    return pl.kernel(body, out_type=tuple(out_type), mesh=mesh,
                     scratch_types=tuple(scratch),
                     compiler_params=pltpu.CompilerParams(
                         use_tc_tiling_on_sc=False))


def _sc_body(h, src_i, dst_i, zeros_s, zeros_d, ones, s_out, deg_out,
             isrc, idst, rows, s_sh, sem, ones_v, deg_sh,
             k_chunks, rows_per_tile, compute_deg):
    c = lax.axis_index("c")
    s = lax.axis_index("s")
    wid = s * NC + c
    row0 = s * rows_per_tile

    pltpu.sync_copy(zeros_s.at[pl.ds(row0, rows_per_tile)],
                    s_sh.at[pl.ds(row0, rows_per_tile)])
    if compute_deg:
        pltpu.sync_copy(zeros_d.at[pl.ds(row0, rows_per_tile)],
                        deg_sh.at[pl.ds(row0, rows_per_tile)])
        pltpu.sync_copy(ones, ones_v)
    plsc.subcore_barrier()

    def step(j, carry):
        pltpu.sync_copy(src_i.at[wid, j], isrc)
        pltpu.sync_copy(dst_i.at[wid, j], idst)
        pltpu.async_copy(h.at[isrc], rows, sem).wait()
        pltpu.sync_copy(rows, s_sh.at[idst], add=True)
        if compute_deg:
            pltpu.sync_copy(ones_v, deg_sh.at[idst], add=True)
        return carry

    lax.fori_loop(0, k_chunks, step, 0)
    plsc.subcore_barrier()

    pltpu.sync_copy(s_sh.at[pl.ds(row0, rows_per_tile)],
                    s_out.at[c, pl.ds(row0, rows_per_tile)])
    if compute_deg:
        pltpu.sync_copy(deg_sh.at[pl.ds(row0, rows_per_tile)],
                        deg_out.at[c, pl.ds(row0, rows_per_tile)])


def _emb_body(x_ref, w_ref, b_ref, o_ref):
    o_ref[...] = jnp.dot(x_ref[...], w_ref[...],
                         preferred_element_type=jnp.float32,
                         precision=lax.Precision.HIGHEST) + b_ref[...]


def _embed(x, w, b, blk):
    n = x.shape[0]
    return pl.pallas_call(
        _emb_body,
        grid=(n // blk,),
        in_specs=[
            pl.BlockSpec((blk, H), lambda i: (i, 0)),
            pl.BlockSpec((H, H), lambda i: (0, 0)),
            pl.BlockSpec((1, H), lambda i: (0, 0)),
        ],
        out_specs=pl.BlockSpec((blk, H), lambda i: (i, 0)),
        out_shape=jax.ShapeDtypeStruct((n, H), jnp.float32),
    )(x, w, b.reshape(1, H))


def _layer_body(h_ref, s_ref, deg_ref, wt_ref, wb_ref, p_ref, o_ref):
    h = h_ref[...]
    s_sum = s_ref[0] + s_ref[1]
    deg = deg_ref[0, :, 0:1] + deg_ref[1, :, 0:1]
    b = p_ref[0:1, :]
    gamma = p_ref[1:2, :]
    beta = p_ref[2:3, :]
    mean = p_ref[3:4, :]
    var = p_ref[4:5, :]
    hw = jnp.dot(h, wb_ref[...], preferred_element_type=jnp.float32,
                 precision=lax.Precision.HIGHEST)
    agg = jnp.dot(s_sum, wt_ref[...], preferred_element_type=jnp.float32,
                  precision=lax.Precision.HIGHEST) + deg * (hw + b)
    z = jax.nn.sigmoid(agg) + jax.nn.softplus(h)
    z = (z - mean) / jnp.sqrt(var + BN_EPS) * gamma + beta
    o_ref[...] = jnp.maximum(z, 0.0)


def _layer_dense(h, s_partial, deg_partial, w, params, blk):
    n = h.shape[0]
    return pl.pallas_call(
        _layer_body,
        grid=(n // blk,),
        in_specs=[
            pl.BlockSpec((blk, H), lambda i: (i, 0)),
            pl.BlockSpec((NC, blk, H), lambda i: (0, i, 0)),
            pl.BlockSpec((NC, blk, L), lambda i: (0, i, 0)),
            pl.BlockSpec((H, H), lambda i: (0, 0)),
            pl.BlockSpec((H, H), lambda i: (0, 0)),
            pl.BlockSpec((5, H), lambda i: (0, 0)),
        ],
        out_specs=pl.BlockSpec((blk, H), lambda i: (i, 0)),
        out_shape=jax.ShapeDtypeStruct((n, H), jnp.float32),
    )(h, s_partial, deg_partial, w[:H], w[H:], params)


def kernel(node_feat, edge_index, W_emb, b_emb,
           w0, b0, gamma0, beta0, mean0, var0,
           w1, b1, gamma1, beta1, mean1, var1):
    n = node_feat.shape[0]
    e = edge_index.shape[1]
    n_pad = ((n + 1 + NS * 8 - 1) // (NS * 8)) * (NS * 8)
    e_pad = ((e + NW * CHUNK - 1) // (NW * CHUNK)) * (NW * CHUNK)
    k_chunks = e_pad // (NW * CHUNK)

    src = edge_index[0].astype(jnp.int32)
    dst = edge_index[1].astype(jnp.int32)
    src_i = jnp.zeros((e_pad,), jnp.int32).at[:e].set(src)
    dst_i = jnp.full((e_pad,), n, jnp.int32).at[:e].set(dst)
    src_i = src_i.reshape(NW, k_chunks, CHUNK)
    dst_i = dst_i.reshape(NW, k_chunks, CHUNK)

    zeros_s = jnp.zeros((n_pad, H), jnp.float32)
    zeros_d = jnp.zeros((n_pad, L), jnp.float32)
    ones = jnp.ones((CHUNK, L), jnp.float32)

    sc_first = _make_sc_segment_sum(n_pad, k_chunks, compute_deg=True)
    sc_next = _make_sc_segment_sum(n_pad, k_chunks, compute_deg=False)

    blk = 2000
    h = _embed(node_feat, W_emb, b_emb, blk)

    s_p, deg_p = sc_first(h, src_i, dst_i, zeros_s, zeros_d, ones)
    params0 = jnp.stack([b0, gamma0, beta0, mean0, var0])
    h = _layer_dense(h, s_p, deg_p, w0, params0, blk)

    (s_p,) = sc_next(h, src_i, dst_i, zeros_s)
    params1 = jnp.stack([b1, gamma1, beta1, mean1, var1])
    h = _layer_dense(h, s_p, deg_p, w1, params1, blk)

    return h
